# baseline (speedup 1.0000x reference)
import jax
import jax.numpy as jnp
from jax.experimental import pallas as pl
from jax.experimental.pallas import tpu as pltpu

_TB = 512
_TOPK_CAP = 32


def _body(k_ref, xt_ref, gate_ref, up_ref, down_ref, o_ref):
    kf = k_ref[0].astype(jnp.float32)
    xt = xt_ref[...]
    z = jnp.dot(gate_ref[...], xt,
                preferred_element_type=jnp.float32)
    g = z * (0.5 + 0.5 * jnp.tanh(0.5 * z))
    bits = jax.lax.bitcast_convert_type(g, jnp.int32) & jnp.int32(0x7FFFFFFF)

    gw = bits.shape[0] // 8
    m = bits[:gw, :]
    for c in range(1, 8):
        m = jnp.maximum(m, bits[c * gw:(c + 1) * gw, :])
    colmax = jnp.max(m, axis=0, keepdims=True)

    def count_ge(data, mid):
        return jnp.sum((data >= mid).astype(jnp.float32), axis=0,
                       keepdims=True)

    lob, hib = jnp.zeros((1, _TB), jnp.int32), colmax + 1
    for _ in range(15):
        mid = lob + jax.lax.shift_right_logical(hib - lob, 1)
        pred = count_ge(m, mid) >= kf
        lob = jnp.where(pred, mid, lob)
        hib = jnp.where(pred, hib, mid)

    cnt_lob = count_ge(bits, lob)
    u = jnp.dot(up_ref[...], xt,
                preferred_element_type=jnp.float32)
    gu = g * u

    def secant_step(lo, hi, cnt_lo, cnt_hi):
        width = (hi - lo).astype(jnp.float32)
        frac = (cnt_lo - kf) / jnp.maximum(cnt_lo - cnt_hi, 1.0)
        step = jnp.floor(width * frac).astype(jnp.int32)
        mid = lo + jnp.clip(step, 1, hi - lo - 1)
        cnt = count_ge(bits, mid)
        pred = cnt >= kf
        return (jnp.where(pred, mid, lo), jnp.where(pred, hi, mid),
                jnp.where(pred, cnt, cnt_lo), jnp.where(pred, cnt_hi, cnt))

    state = (lob, colmax + 1, cnt_lob, jnp.zeros_like(cnt_lob))

    def bs_cond(carry):
        lo, hi, cnt_lo, cnt_hi = carry
        open_ = jnp.where(cnt_lo == kf, 0, hi - lo)
        return jnp.max(open_) > 1

    def bs(carry):
        return secant_step(*secant_step(*secant_step(*carry)))

    lo, _, _, _ = jax.lax.while_loop(bs_cond, bs, state)

    h = jnp.where(bits >= lo, gu, 0.0).astype(jnp.bfloat16)
    o_ref[...] = jnp.dot(down_ref[...], h,
                         preferred_element_type=jnp.float32)


def kernel(x, gate_all, up_all, down_all, layer_idx, top_k):
    gate = jax.lax.dynamic_index_in_dim(gate_all, layer_idx, 0, keepdims=False)
    up = jax.lax.dynamic_index_in_dim(up_all, layer_idx, 0, keepdims=False)
    down = jax.lax.dynamic_index_in_dim(down_all, layer_idx, 0, keepdims=False)
    b, s, d = x.shape
    ii = gate.shape[0]
    xt = x.reshape(s, d).T
    k_eff = jnp.minimum(jnp.asarray(top_k, jnp.int32), _TOPK_CAP).reshape(1)

    out_t = pl.pallas_call(
        _body,
        grid=(s // _TB,),
        out_shape=jax.ShapeDtypeStruct((d, s), jnp.float32),
        in_specs=[
            pl.BlockSpec(memory_space=pltpu.SMEM),
            pl.BlockSpec((d, _TB), lambda i: (0, i)),
            pl.BlockSpec((ii, d), lambda i: (0, 0)),
            pl.BlockSpec((ii, d), lambda i: (0, 0)),
            pl.BlockSpec((d, ii), lambda i: (0, 0)),
        ],
        out_specs=pl.BlockSpec((d, _TB), lambda i: (0, i)),
        compiler_params=pltpu.CompilerParams(
            vmem_limit_bytes=110 * 1024 * 1024,
        ),
    )(k_eff, xt.astype(jnp.bfloat16), gate.astype(jnp.bfloat16),
      up.astype(jnp.bfloat16), down.astype(jnp.bfloat16))
    return out_t.T.reshape(b, s, d)


# speedup vs baseline: 1.0108x; 1.0108x over previous
import jax
import jax.numpy as jnp
from jax.experimental import pallas as pl
from jax.experimental.pallas import tpu as pltpu

_TB = 512
_TOPK_CAP = 32


def _body(k_ref, xt_ref, gate_ref, up_ref, down_ref, o_ref):
    kf = k_ref[0].astype(jnp.float32)
    xt = xt_ref[...]
    z = jnp.dot(gate_ref[...], xt,
                preferred_element_type=jnp.float32)
    g = z * (0.5 + 0.5 * jnp.tanh(0.5 * z))
    bits = jax.lax.bitcast_convert_type(g, jnp.int32) & jnp.int32(0x7FFFFFFF)

    gw = bits.shape[0] // 8
    m = bits[:gw, :]
    for c in range(1, 8):
        m = jnp.maximum(m, bits[c * gw:(c + 1) * gw, :])
    colmax = jnp.max(m, axis=0, keepdims=True)

    def count_ge(data, mid):
        return jnp.sum((data >= mid).astype(jnp.float32), axis=0,
                       keepdims=True)

    lob, hib = jnp.zeros((1, _TB), jnp.int32), colmax + 1
    for _ in range(15):
        mid = lob + jax.lax.shift_right_logical(hib - lob, 1)
        pred = count_ge(m, mid) >= kf
        lob = jnp.where(pred, mid, lob)
        hib = jnp.where(pred, hib, mid)

    cnt_lob = count_ge(bits, lob)
    u = jnp.dot(up_ref[...], xt,
                preferred_element_type=jnp.float32)
    gu = g * u

    def secant_step(lo, hi, cnt_lo, cnt_hi):
        width = (hi - lo).astype(jnp.float32)
        frac = (cnt_lo - kf) / jnp.maximum(cnt_lo - cnt_hi, 1.0)
        step = jnp.floor(width * frac).astype(jnp.int32)
        mid = lo + jnp.clip(step, 1, hi - lo - 1)
        cnt = count_ge(bits, mid)
        pred = cnt >= kf
        return (jnp.where(pred, mid, lo), jnp.where(pred, hi, mid),
                jnp.where(pred, cnt, cnt_lo), jnp.where(pred, cnt_hi, cnt))

    state = (lob, colmax + 1, cnt_lob, jnp.zeros_like(cnt_lob))

    def bs_cond(carry):
        lo, hi, cnt_lo, cnt_hi = carry
        open_ = jnp.where(cnt_lo == kf, 0, hi - lo)
        return jnp.max(open_) > 1

    def bs(carry):
        return secant_step(*secant_step(*carry))

    lo, _, _, _ = jax.lax.while_loop(bs_cond, bs, state)

    h = jnp.where(bits >= lo, gu, 0.0).astype(jnp.bfloat16)
    o_ref[...] = jnp.dot(down_ref[...], h,
                         preferred_element_type=jnp.float32)


def kernel(x, gate_all, up_all, down_all, layer_idx, top_k):
    gate = jax.lax.dynamic_index_in_dim(gate_all, layer_idx, 0, keepdims=False)
    up = jax.lax.dynamic_index_in_dim(up_all, layer_idx, 0, keepdims=False)
    down = jax.lax.dynamic_index_in_dim(down_all, layer_idx, 0, keepdims=False)
    b, s, d = x.shape
    ii = gate.shape[0]
    xt = x.reshape(s, d).T
    k_eff = jnp.minimum(jnp.asarray(top_k, jnp.int32), _TOPK_CAP).reshape(1)

    out_t = pl.pallas_call(
        _body,
        grid=(s // _TB,),
        out_shape=jax.ShapeDtypeStruct((d, s), jnp.float32),
        in_specs=[
            pl.BlockSpec(memory_space=pltpu.SMEM),
            pl.BlockSpec((d, _TB), lambda i: (0, i)),
            pl.BlockSpec((ii, d), lambda i: (0, 0)),
            pl.BlockSpec((ii, d), lambda i: (0, 0)),
            pl.BlockSpec((d, ii), lambda i: (0, 0)),
        ],
        out_specs=pl.BlockSpec((d, _TB), lambda i: (0, i)),
        compiler_params=pltpu.CompilerParams(
            vmem_limit_bytes=110 * 1024 * 1024,
        ),
    )(k_eff, xt.astype(jnp.bfloat16), gate.astype(jnp.bfloat16),
      up.astype(jnp.bfloat16), down.astype(jnp.bfloat16))
    return out_t.T.reshape(b, s, d)
